# baseline (speedup 1.0000x reference)
import jax
import jax.numpy as jnp
from jax.experimental import pallas as pl

EPS = 1e-05
LOG2E = 1.4426950408889634
CAP = 80.0
BLOCK_B = 16384


def _body(x_ref, c_ref, wt_ref, o_ref):
    x = x_ref[...]
    c = c_ref[...]
    d = LOG2E / (jnp.abs(x - c) + EPS)
    u = jnp.exp2(jnp.minimum(d, CAP)).astype(jnp.bfloat16)
    vs = jnp.dot(wt_ref[...], u, preferred_element_type=jnp.float32)
    embed_dim = vs.shape[0] - 1
    o_ref[...] = vs[:embed_dim, :] * (1.0 / vs[embed_dim:, :])


def kernel(x, emb_weight, centroid):
    batch = x.shape[0]
    num_level, embed_dim = emb_weight.shape
    x_row = x.reshape(1, batch)
    w_aug_t = jnp.concatenate(
        [emb_weight.T, jnp.ones((1, num_level), jnp.float32)],
        axis=0).astype(jnp.bfloat16)
    grid = batch // BLOCK_B
    out_t = pl.pallas_call(
        _body,
        grid=(grid,),
        in_specs=[
            pl.BlockSpec((1, BLOCK_B), lambda i: (0, i)),
            pl.BlockSpec((num_level, 1), lambda i: (0, 0)),
            pl.BlockSpec((embed_dim + 1, num_level), lambda i: (0, 0)),
        ],
        out_specs=pl.BlockSpec((embed_dim, BLOCK_B), lambda i: (0, i)),
        out_shape=jax.ShapeDtypeStruct((embed_dim, batch), jnp.float32),
    )(x_row, centroid, w_aug_t)
    return out_t.T
